# baseline (speedup 1.0000x reference)
import functools

import jax
import jax.numpy as jnp
from jax import lax
from jax.experimental import pallas as pl
from jax.experimental.pallas import tpu as pltpu
from jax.experimental.pallas import tpu_sc as plsc

_NC = 2
_NS = 16
_NW = _NC * _NS

_ANCHOR_W = (1.08, 3.42, 6.63)
_ANCHOR_H = (1.19, 4.41, 11.38)


def _decode_pairs(x3, consts, grid_tab, n_pairs, g, n_ch):
    npos = g * g
    pairs_per_tec = n_pairs // _NW
    n_chunks = 19
    cpos = npos // n_chunks
    cvecs = cpos // 16
    cwords = cpos * n_ch
    mesh = plsc.VectorSubcoreMesh(
        core_axis_name="c", subcore_axis_name="s",
        num_cores=_NC, num_subcores=_NS)

    @functools.partial(
        pl.kernel,
        out_type=jax.ShapeDtypeStruct((n_pairs, npos * n_ch), jnp.float32),
        mesh=mesh,
        compiler_params=pltpu.CompilerParams(
            needs_layout_passes=False, use_tc_tiling_on_sc=False),
        scratch_types=[
            pltpu.VMEM((n_ch, npos), jnp.float32),
            pltpu.VMEM((cwords,), jnp.float32),
            pltpu.VMEM((2, npos), jnp.float32),
            pltpu.VMEM((3, 16), jnp.float32),
        ],
    )
    def run(x_hbm, consts_hbm, grid_hbm, out_hbm, in_v, out_v, tab_v, cv):
        wid = lax.axis_index("s") * _NC + lax.axis_index("c")
        lane10 = lax.iota(jnp.int32, 16) * 10
        pltpu.sync_copy(grid_hbm, tab_v)

        def do_pair(pp, carry):
            pair = wid * pairs_per_tec + pp
            a = lax.rem(pair, 3)
            pltpu.sync_copy(consts_hbm.at[a], cv)
            pltpu.sync_copy(x_hbm.at[pair], in_v)
            stride_v = cv[0, :]
            aw_v = cv[1, :]
            ah_v = cv[2, :]

            def do_chunk(k, c1):
                def do_vec(vl, c2):
                    p0 = k * cpos + vl * 16
                    jf = tab_v[0, pl.ds(p0, 16)]
                    if_ = tab_v[1, pl.ds(p0, 16)]
                    idx0 = lane10 + vl * (16 * n_ch)
                    for c in range(n_ch):
                        t = in_v[c, pl.ds(p0, 16)]
                        if c == 0:
                            r = (1.0 / (1.0 + jnp.exp(-t)) + jf) * stride_v
                        elif c == 1:
                            r = (1.0 / (1.0 + jnp.exp(-t)) + if_) * stride_v
                        elif c == 2:
                            r = jnp.minimum(jnp.exp(t), 1000.0) * aw_v
                        elif c == 3:
                            r = jnp.minimum(jnp.exp(t), 1000.0) * ah_v
                        elif c in (4, 5):
                            r = t
                        else:
                            r = 1.0 / (1.0 + jnp.exp(-t))
                        plsc.store_scatter(out_v, [idx0 + c], r)
                    return c2

                lax.fori_loop(0, cvecs, do_vec, 0, unroll=False)
                pltpu.sync_copy(out_v,
                                out_hbm.at[pair, pl.ds(k * cwords, cwords)])
                return c1

            lax.fori_loop(0, n_chunks, do_chunk, 0, unroll=False)
            return carry

        lax.fori_loop(0, pairs_per_tec, do_pair, 0, unroll=False)

    return run(x3, consts, grid_tab)


def kernel(x, targets, img_size):
    del targets
    num_samples, cin, g, g2 = x.shape
    assert g == g2
    n_ch = 10
    n_anchors = cin // n_ch
    npos = g * g
    n_pairs = num_samples * n_anchors

    stride = jnp.asarray(img_size, jnp.float32) / g
    aw = jnp.asarray(_ANCHOR_W, jnp.float32)
    ah = jnp.asarray(_ANCHOR_H, jnp.float32)
    consts = jnp.stack(
        [jnp.broadcast_to(stride, (3,)), aw, ah], axis=1
    )[:, :, None] * jnp.ones((1, 1, 16), jnp.float32)

    pos = jnp.arange(npos, dtype=jnp.int32)
    grid_tab = jnp.stack(
        [(pos % g).astype(jnp.float32), (pos // g).astype(jnp.float32)]
    )

    x3 = x.reshape(n_pairs, n_ch, npos)
    out = _decode_pairs(x3, consts, grid_tab, n_pairs, g, n_ch)
    return out.reshape(num_samples, n_anchors * npos, n_ch)


# speedup vs baseline: 1.0282x; 1.0282x over previous
import functools

import jax
import jax.numpy as jnp
from jax import lax
from jax.experimental import pallas as pl
from jax.experimental.pallas import tpu as pltpu
from jax.experimental.pallas import tpu_sc as plsc

_NC = 2
_NS = 16
_NW = _NC * _NS

_ANCHOR_W = (1.08, 3.42, 6.63)
_ANCHOR_H = (1.19, 4.41, 11.38)


def _decode_pairs(x3, consts, n_pairs, g, n_ch):
    npos = g * g
    pairs_per_tec = n_pairs // _NW
    n_chunks = 19
    cpos = npos // n_chunks
    cvecs = cpos // 16
    cwords = cpos * n_ch
    mesh = plsc.VectorSubcoreMesh(
        core_axis_name="c", subcore_axis_name="s",
        num_cores=_NC, num_subcores=_NS)

    @functools.partial(
        pl.kernel,
        out_type=jax.ShapeDtypeStruct((n_pairs, npos * n_ch), jnp.float32),
        mesh=mesh,
        compiler_params=pltpu.CompilerParams(
            needs_layout_passes=False, use_tc_tiling_on_sc=False),
        scratch_types=[
            pltpu.VMEM((2, n_ch, npos), jnp.float32),
            pltpu.VMEM((2, cwords), jnp.float32),
            pltpu.VMEM((3, 3, 16), jnp.float32),
            pltpu.SemaphoreType.DMA((2,)),
            pltpu.SemaphoreType.DMA((2,)),
        ],
    )
    def run(x_hbm, consts_hbm, out_hbm, in_v, out_v, cv, insem, outsem):
        wid = lax.axis_index("s") * _NC + lax.axis_index("c")
        lane = lax.iota(jnp.int32, 16)
        lane10 = lane * 10
        g16 = jnp.full((16,), g, jnp.int32)
        pltpu.sync_copy(consts_hbm, cv)
        first = wid * pairs_per_tec
        pltpu.async_copy(x_hbm.at[first], in_v.at[0], insem.at[0])

        for s in range(pairs_per_tec):
            par = s % 2
            pair = first + s
            a = lax.rem(pair, 3)
            pltpu.make_async_copy(x_hbm.at[pair], in_v.at[par],
                                  insem.at[par]).wait()
            if s + 1 < pairs_per_tec:
                pltpu.async_copy(x_hbm.at[pair + 1], in_v.at[1 - par],
                                 insem.at[1 - par])
            stride_v = cv[a, 0, :]
            aw_v = cv[a, 1, :]
            ah_v = cv[a, 2, :]
            in_s = in_v.at[par]

            def do_chunk(k, c1, in_s=in_s, pair=pair, stride_v=stride_v,
                         aw_v=aw_v, ah_v=ah_v):
                kp = lax.rem(k, 2)
                obuf = out_v.at[kp]

                @pl.when(k >= 2)
                def _wait_prev():
                    pltpu.make_async_copy(
                        obuf, out_hbm.at[pair, pl.ds((k - 2) * cwords, cwords)],
                        outsem.at[kp]).wait()

                def do_vec(vl, c2):
                    p0 = k * cpos + vl * 16
                    pos = p0 + lane
                    jf = lax.rem(pos, g16).astype(jnp.float32)
                    if_ = lax.div(pos, g16).astype(jnp.float32)
                    idx0 = lane10 + vl * (16 * n_ch)
                    for c in range(n_ch):
                        t = in_s[c, pl.ds(p0, 16)]
                        if c == 0:
                            r = (1.0 / (1.0 + jnp.exp(-t)) + jf) * stride_v
                        elif c == 1:
                            r = (1.0 / (1.0 + jnp.exp(-t)) + if_) * stride_v
                        elif c == 2:
                            r = jnp.minimum(jnp.exp(t), 1000.0) * aw_v
                        elif c == 3:
                            r = jnp.minimum(jnp.exp(t), 1000.0) * ah_v
                        elif c in (4, 5):
                            r = t
                        else:
                            r = 1.0 / (1.0 + jnp.exp(-t))
                        plsc.store_scatter(obuf, [idx0 + c], r)
                    return c2

                lax.fori_loop(0, cvecs, do_vec, 0, unroll=False)
                pltpu.async_copy(obuf,
                                 out_hbm.at[pair, pl.ds(k * cwords, cwords)],
                                 outsem.at[kp])
                return c1

            lax.fori_loop(0, n_chunks, do_chunk, 0, unroll=False)
            for kk in (n_chunks - 2, n_chunks - 1):
                pltpu.make_async_copy(
                    out_v.at[kk % 2],
                    out_hbm.at[pair, pl.ds(kk * cwords, cwords)],
                    outsem.at[kk % 2]).wait()

    return run(x3, consts)


def kernel(x, targets, img_size):
    del targets
    num_samples, cin, g, g2 = x.shape
    assert g == g2
    n_ch = 10
    n_anchors = cin // n_ch
    npos = g * g
    n_pairs = num_samples * n_anchors

    stride = jnp.asarray(img_size, jnp.float32) / g
    aw = jnp.asarray(_ANCHOR_W, jnp.float32)
    ah = jnp.asarray(_ANCHOR_H, jnp.float32)
    consts = jnp.stack(
        [jnp.broadcast_to(stride, (3,)), aw, ah], axis=1
    )[:, :, None] * jnp.ones((1, 1, 16), jnp.float32)

    x3 = x.reshape(n_pairs, n_ch, npos)
    out = _decode_pairs(x3, consts, n_pairs, g, n_ch)
    return out.reshape(num_samples, n_anchors * npos, n_ch)


# speedup vs baseline: 14.1606x; 13.7720x over previous
import jax
import jax.numpy as jnp
from jax import lax
from jax.experimental import pallas as pl
from jax.experimental.pallas import tpu as pltpu

_ANCHOR_W = (1.08, 3.42, 6.63)
_ANCHOR_H = (1.19, 4.41, 11.38)


def _make_body(c, g, n_anchors):
    def body(stride_ref, *refs):
        o_ref = refs[-1]
        if c == 0:
            jvec = lax.broadcasted_iota(jnp.int32, (8, g), 1).astype(jnp.float32)
        for a in range(n_anchors):
            x_ref = refs[a]
            if c in (0, 1):
                scale = stride_ref[0]
            elif c == 2:
                scale = _ANCHOR_W[a]
            elif c == 3:
                scale = _ANCHOR_H[a]
            for i in range(g):
                t = x_ref[i]
                if c == 0:
                    r = (1.0 / (1.0 + jnp.exp(-t)) + jvec) * scale
                elif c == 1:
                    r = (1.0 / (1.0 + jnp.exp(-t)) + jnp.float32(i)) * scale
                elif c in (2, 3):
                    r = jnp.minimum(jnp.exp(t), 1000.0) * scale
                elif c in (4, 5):
                    r = t
                else:
                    r = 1.0 / (1.0 + jnp.exp(-t))
                o_ref[:, pl.ds((a * g + i) * g, g)] = r

    return body


def _decode_channel(c, x_t, prev, stride1, g, n_anchors, n_ch):
    nb = x_t.shape[2]
    npos = n_anchors * g * g
    grid = (nb // 8,)

    def in_map(a, c=c):
        return lambda bb, a=a, c=c: (a * n_ch + c, 0, bb, 0)

    in_specs = [pl.BlockSpec(memory_space=pltpu.SMEM)]
    in_specs += [
        pl.BlockSpec((None, g, 8, g), in_map(a)) for a in range(n_anchors)
    ]
    operands = [stride1] + [x_t] * n_anchors
    aliases = {}
    if prev is not None:
        in_specs.append(pl.BlockSpec(memory_space=pltpu.HBM))
        operands.append(prev)
        aliases = {1 + n_anchors: 0}

    return pl.pallas_call(
        _make_body(c, g, n_anchors),
        grid=grid,
        in_specs=in_specs,
        out_specs=pl.BlockSpec(
            (None, 8, npos),
            lambda bb, c=c: (c, bb, 0),
        ),
        out_shape=jax.ShapeDtypeStruct((n_ch, nb, npos), jnp.float32),
        input_output_aliases=aliases,
        compiler_params=pltpu.CompilerParams(
            dimension_semantics=("arbitrary",),
        ),
    )(*operands)


def kernel(x, targets, img_size):
    del targets
    num_samples, cin, g, g2 = x.shape
    assert g == g2
    n_ch = 10
    n_anchors = cin // n_ch

    stride1 = (jnp.asarray(img_size, jnp.float32) / g).reshape(1)

    x_t = jnp.transpose(x, (1, 2, 0, 3))
    out_t = None
    for c in range(n_ch):
        out_t = _decode_channel(c, x_t, out_t, stride1, g, n_anchors, n_ch)
    return jnp.transpose(out_t, (1, 2, 0))


# speedup vs baseline: 16.2104x; 1.1448x over previous
import jax
import jax.numpy as jnp
from jax import lax
from jax.experimental import pallas as pl
from jax.experimental.pallas import tpu as pltpu

_ANCHOR_W = (1.08, 3.42, 6.63)
_ANCHOR_H = (1.19, 4.41, 11.38)


def _make_body(g, n_anchors, n_ch):
    def chan_body(c, stride_ref, x_refs, o_ref):
        if c == 0:
            jvec = lax.broadcasted_iota(
                jnp.int32, (8, g), 1).astype(jnp.float32)
        for a in range(n_anchors):
            x_ref = x_refs[a]
            if c in (0, 1):
                scale = stride_ref[0]
            elif c == 2:
                scale = _ANCHOR_W[a]
            elif c == 3:
                scale = _ANCHOR_H[a]
            for i in range(g):
                t = x_ref[i]
                if c == 0:
                    r = (1.0 / (1.0 + jnp.exp(-t)) + jvec) * scale
                elif c == 1:
                    r = (1.0 / (1.0 + jnp.exp(-t)) + jnp.float32(i)) * scale
                elif c in (2, 3):
                    r = jnp.minimum(jnp.exp(t), 1000.0) * scale
                elif c in (4, 5):
                    r = t
                else:
                    r = 1.0 / (1.0 + jnp.exp(-t))
                o_ref[:, pl.ds((a * g + i) * g, g)] = r

    def body(stride_ref, *refs):
        o_ref = refs[-1]
        x_refs = refs[:-1]
        c_idx = pl.program_id(0)
        for c in range(n_ch):
            if c == 5:
                continue
            cond = (c_idx == c) if c != 4 else (
                jnp.logical_or(c_idx == 4, c_idx == 5))

            @pl.when(cond)
            def _(c=c):
                chan_body(c, stride_ref, x_refs, o_ref)

    return body


def _decode(x_t, stride1, g, n_anchors, n_ch):
    nb = x_t.shape[2]
    npos = n_anchors * g * g
    grid = (n_ch, nb // 8)

    def in_map(a):
        return lambda c, bb, a=a: (a * n_ch + c, 0, bb, 0)

    in_specs = [pl.BlockSpec(memory_space=pltpu.SMEM)]
    in_specs += [
        pl.BlockSpec((None, g, 8, g), in_map(a)) for a in range(n_anchors)
    ]

    return pl.pallas_call(
        _make_body(g, n_anchors, n_ch),
        grid=grid,
        in_specs=in_specs,
        out_specs=pl.BlockSpec(
            (None, 8, npos),
            lambda c, bb: (c, bb, 0),
        ),
        out_shape=jax.ShapeDtypeStruct((n_ch, nb, npos), jnp.float32),
        compiler_params=pltpu.CompilerParams(
            dimension_semantics=("arbitrary", "arbitrary"),
        ),
    )(stride1, *([x_t] * n_anchors))


def kernel(x, targets, img_size):
    del targets
    num_samples, cin, g, g2 = x.shape
    assert g == g2
    n_ch = 10
    n_anchors = cin // n_ch

    stride1 = (jnp.asarray(img_size, jnp.float32) / g).reshape(1)

    x_t = jnp.transpose(x, (1, 2, 0, 3))
    out_t = _decode(x_t, stride1, g, n_anchors, n_ch)
    return jnp.transpose(out_t, (1, 2, 0))
